# baseline (device time: 85751 ns/iter reference)
import jax
import jax.numpy as jnp
from jax import lax
from jax.experimental import pallas as pl
from jax.experimental.pallas import tpu as pltpu

N_DEV = 4
M = 1024
N = 1024
CH = M // N_DEV


def kernel(x, w_mat):
    def body(x_ref, w_ref, out_ref, comm_ref, send_sems, recv_sems):
        my = lax.axis_index("i")
        left = jnp.remainder(my - 1 + N_DEV, N_DEV)
        right = jnp.remainder(my + 1, N_DEV)

        barrier_sem = pltpu.get_barrier_semaphore()
        for nbr in (left, right):
            pl.semaphore_signal(
                barrier_sem, inc=1,
                device_id=(nbr,), device_id_type=pl.DeviceIdType.MESH,
            )
        pl.semaphore_wait(barrier_sem, 2)

        out_ref[:, :] = jnp.dot(
            x_ref[:, :], w_ref[:, :], preferred_element_type=jnp.float32
        )

        for s in range(N_DEV - 1):
            send_idx = jnp.remainder(my - s + N_DEV, N_DEV)
            recv_idx = jnp.remainder(my - s - 1 + N_DEV, N_DEV)
            rdma = pltpu.make_async_remote_copy(
                src_ref=out_ref.at[pl.ds(send_idx * CH, CH), :],
                dst_ref=comm_ref.at[s],
                send_sem=send_sems.at[s],
                recv_sem=recv_sems.at[s],
                device_id=(right,),
                device_id_type=pl.DeviceIdType.MESH,
            )
            rdma.start()
            rdma.wait()
            out_ref[pl.ds(recv_idx * CH, CH), :] += comm_ref[s]

        for s in range(N_DEV - 1):
            idx = jnp.remainder(my + 1 - s + N_DEV, N_DEV)
            rdma = pltpu.make_async_remote_copy(
                src_ref=out_ref.at[pl.ds(idx * CH, CH), :],
                dst_ref=out_ref.at[pl.ds(idx * CH, CH), :],
                send_sem=send_sems.at[N_DEV - 1 + s],
                recv_sem=recv_sems.at[N_DEV - 1 + s],
                device_id=(right,),
                device_id_type=pl.DeviceIdType.MESH,
            )
            rdma.start()
            rdma.wait()

    n_sems = 2 * (N_DEV - 1)
    return pl.pallas_call(
        body,
        out_shape=jax.ShapeDtypeStruct((M, N), jnp.float32),
        in_specs=[
            pl.BlockSpec(memory_space=pltpu.VMEM),
            pl.BlockSpec(memory_space=pltpu.VMEM),
        ],
        out_specs=pl.BlockSpec(memory_space=pltpu.VMEM),
        scratch_shapes=[
            pltpu.VMEM((N_DEV - 1, CH, N), jnp.float32),
            pltpu.SemaphoreType.DMA((n_sems,)),
            pltpu.SemaphoreType.DMA((n_sems,)),
        ],
        compiler_params=pltpu.CompilerParams(collective_id=0),
    )(x, w_mat)


# device time: 48820 ns/iter; 1.7565x vs baseline; 1.7565x over previous
import jax
import jax.numpy as jnp
from jax import lax
from jax.experimental import pallas as pl
from jax.experimental.pallas import tpu as pltpu

N_DEV = 4
M = 1024
N = 1024
H = M // 2
Q = M // 4
C = N // 2


def kernel(x, w_mat):
    def body(x_ref, w_ref, out_ref, c1_ref, c2_ref, send_sems, recv_sems):
        my = lax.axis_index("i")
        pa = my ^ 1
        pb = 3 - my
        bit_a = (my ^ (my >> 1)) & 1
        bit_b = (my >> 1) & 1
        bit1a = my & 1

        barrier_sem = pltpu.get_barrier_semaphore()
        for nbr in (pa, pb):
            pl.semaphore_signal(
                barrier_sem, inc=1,
                device_id=(nbr,), device_id_type=pl.DeviceIdType.MESH,
            )
        pl.semaphore_wait(barrier_sem, 2)

        out_ref[:, :] = jnp.dot(
            x_ref[:, :], w_ref[:, :], preferred_element_type=jnp.float32
        )

        def copy(src, dst, sem_idx, peer):
            return pltpu.make_async_remote_copy(
                src_ref=src, dst_ref=dst,
                send_sem=send_sems.at[sem_idx],
                recv_sem=recv_sems.at[sem_idx],
                device_id=(peer,),
                device_id_type=pl.DeviceIdType.MESH,
            )

        r0 = copy(out_ref.at[pl.ds((1 - bit_a) * H, H), pl.ds(0, C)],
                  c1_ref.at[0], 0, pa)
        r1 = copy(out_ref.at[pl.ds((1 - bit_b) * H, H), pl.ds(C, C)],
                  c1_ref.at[1], 1, pb)
        r0.start()
        r1.start()
        r0.wait()
        r1.wait()
        out_ref[pl.ds(bit_a * H, H), pl.ds(0, C)] += c1_ref[0]
        out_ref[pl.ds(bit_b * H, H), pl.ds(C, C)] += c1_ref[1]

        q_keep0 = 2 * bit_a + bit_b
        q_send0 = 2 * bit_a + (1 - bit_b)
        q_keep1 = 2 * bit_b + bit1a
        q_send1 = 2 * bit_b + (1 - bit1a)
        r0 = copy(out_ref.at[pl.ds(q_send0 * Q, Q), pl.ds(0, C)],
                  c2_ref.at[0], 2, pb)
        r1 = copy(out_ref.at[pl.ds(q_send1 * Q, Q), pl.ds(C, C)],
                  c2_ref.at[1], 3, pa)
        r0.start()
        r1.start()
        r0.wait()
        r1.wait()
        out_ref[pl.ds(q_keep0 * Q, Q), pl.ds(0, C)] += c2_ref[0]
        out_ref[pl.ds(q_keep1 * Q, Q), pl.ds(C, C)] += c2_ref[1]

        r0 = copy(out_ref.at[pl.ds(q_keep0 * Q, Q), pl.ds(0, C)],
                  out_ref.at[pl.ds(q_keep0 * Q, Q), pl.ds(0, C)], 4, pb)
        r1 = copy(out_ref.at[pl.ds(q_keep1 * Q, Q), pl.ds(C, C)],
                  out_ref.at[pl.ds(q_keep1 * Q, Q), pl.ds(C, C)], 5, pa)
        r0.start()
        r1.start()
        r0.wait()
        r1.wait()

        r0 = copy(out_ref.at[pl.ds(bit_a * H, H), pl.ds(0, C)],
                  out_ref.at[pl.ds(bit_a * H, H), pl.ds(0, C)], 6, pa)
        r1 = copy(out_ref.at[pl.ds(bit_b * H, H), pl.ds(C, C)],
                  out_ref.at[pl.ds(bit_b * H, H), pl.ds(C, C)], 7, pb)
        r0.start()
        r1.start()
        r0.wait()
        r1.wait()

    return pl.pallas_call(
        body,
        out_shape=jax.ShapeDtypeStruct((M, N), jnp.float32),
        in_specs=[
            pl.BlockSpec(memory_space=pltpu.VMEM),
            pl.BlockSpec(memory_space=pltpu.VMEM),
        ],
        out_specs=pl.BlockSpec(memory_space=pltpu.VMEM),
        scratch_shapes=[
            pltpu.VMEM((2, H, C), jnp.float32),
            pltpu.VMEM((2, Q, C), jnp.float32),
            pltpu.SemaphoreType.DMA((8,)),
            pltpu.SemaphoreType.DMA((8,)),
        ],
        compiler_params=pltpu.CompilerParams(collective_id=0),
    )(x, w_mat)


# device time: 48469 ns/iter; 1.7692x vs baseline; 1.0072x over previous
import jax
import jax.numpy as jnp
from jax import lax
from jax.experimental import pallas as pl
from jax.experimental.pallas import tpu as pltpu

N_DEV = 4
M = 1024
N = 1024
H = M // 2
Q = M // 4
C = N // 2


def kernel(x, w_mat):
    def body(x_ref, w_ref, out_ref, c1_ref, c2_ref, send_sems, recv_sems):
        my = lax.axis_index("i")
        pa = my ^ 1
        pb = 3 - my
        bit_a = (my ^ (my >> 1)) & 1
        bit_b = (my >> 1) & 1
        bit1a = my & 1

        barrier_sem = pltpu.get_barrier_semaphore()
        for nbr in (pa, pb):
            pl.semaphore_signal(
                barrier_sem, inc=1,
                device_id=(nbr,), device_id_type=pl.DeviceIdType.MESH,
            )
        pl.semaphore_wait(barrier_sem, 2)

        def gemm_block(rh, cs):
            out_ref[pl.ds(rh * H, H), pl.ds(cs * C, C)] = jnp.dot(
                x_ref[pl.ds(rh * H, H), :],
                w_ref[:, pl.ds(cs * C, C)],
                preferred_element_type=jnp.float32,
            )

        def copy(src, dst, sem_idx, peer):
            return pltpu.make_async_remote_copy(
                src_ref=src, dst_ref=dst,
                send_sem=send_sems.at[sem_idx],
                recv_sem=recv_sems.at[sem_idx],
                device_id=(peer,),
                device_id_type=pl.DeviceIdType.MESH,
            )

        gemm_block(1 - bit_a, 0)
        gemm_block(1 - bit_b, 1)

        p1s0 = copy(out_ref.at[pl.ds((1 - bit_a) * H, H), pl.ds(0, C)],
                    c1_ref.at[0], 0, pa)
        p1s1 = copy(out_ref.at[pl.ds((1 - bit_b) * H, H), pl.ds(C, C)],
                    c1_ref.at[1], 1, pb)
        p1s0.start()
        p1s1.start()

        gemm_block(bit_a, 0)
        gemm_block(bit_b, 1)

        q_keep0 = 2 * bit_a + bit_b
        q_send0 = 2 * bit_a + (1 - bit_b)
        q_keep1 = 2 * bit_b + bit1a
        q_send1 = 2 * bit_b + (1 - bit1a)

        p1s0.wait()
        out_ref[pl.ds(bit_a * H, H), pl.ds(0, C)] += c1_ref[0]
        p2s0 = copy(out_ref.at[pl.ds(q_send0 * Q, Q), pl.ds(0, C)],
                    c2_ref.at[0], 2, pb)
        p2s0.start()

        p1s1.wait()
        out_ref[pl.ds(bit_b * H, H), pl.ds(C, C)] += c1_ref[1]
        p2s1 = copy(out_ref.at[pl.ds(q_send1 * Q, Q), pl.ds(C, C)],
                    c2_ref.at[1], 3, pa)
        p2s1.start()

        p2s0.wait()
        out_ref[pl.ds(q_keep0 * Q, Q), pl.ds(0, C)] += c2_ref[0]
        p3s0 = copy(out_ref.at[pl.ds(q_keep0 * Q, Q), pl.ds(0, C)],
                    out_ref.at[pl.ds(q_keep0 * Q, Q), pl.ds(0, C)], 4, pb)
        p3s0.start()

        p2s1.wait()
        out_ref[pl.ds(q_keep1 * Q, Q), pl.ds(C, C)] += c2_ref[1]
        p3s1 = copy(out_ref.at[pl.ds(q_keep1 * Q, Q), pl.ds(C, C)],
                    out_ref.at[pl.ds(q_keep1 * Q, Q), pl.ds(C, C)], 5, pa)
        p3s1.start()

        p3s0.wait()
        p4s0 = copy(out_ref.at[pl.ds(bit_a * H, H), pl.ds(0, C)],
                    out_ref.at[pl.ds(bit_a * H, H), pl.ds(0, C)], 6, pa)
        p4s0.start()

        p3s1.wait()
        p4s1 = copy(out_ref.at[pl.ds(bit_b * H, H), pl.ds(C, C)],
                    out_ref.at[pl.ds(bit_b * H, H), pl.ds(C, C)], 7, pb)
        p4s1.start()

        p4s0.wait()
        p4s1.wait()

    return pl.pallas_call(
        body,
        out_shape=jax.ShapeDtypeStruct((M, N), jnp.float32),
        in_specs=[
            pl.BlockSpec(memory_space=pltpu.VMEM),
            pl.BlockSpec(memory_space=pltpu.VMEM),
        ],
        out_specs=pl.BlockSpec(memory_space=pltpu.VMEM),
        scratch_shapes=[
            pltpu.VMEM((2, H, C), jnp.float32),
            pltpu.VMEM((2, Q, C), jnp.float32),
            pltpu.SemaphoreType.DMA((8,)),
            pltpu.SemaphoreType.DMA((8,)),
        ],
        compiler_params=pltpu.CompilerParams(collective_id=0),
    )(x, w_mat)


# device time: 43796 ns/iter; 1.9580x vs baseline; 1.1067x over previous
import jax
import jax.numpy as jnp
from jax import lax
from jax.experimental import pallas as pl
from jax.experimental.pallas import tpu as pltpu

N_DEV = 4
M = 1024
N = 1024
H = M // 2
Q = M // 4
SC = N // 4
ORDER = (0, 2, 1, 3)


def kernel(x, w_mat):
    def body(x_ref, w_ref, out_ref, c1_ref, c2_ref, send_sems, recv_sems):
        my = lax.axis_index("i")
        pa = my ^ 1
        pb = 3 - my
        bit_a = (my ^ (my >> 1)) & 1
        bit_b = (my >> 1) & 1
        bit1a = my & 1

        chains = []
        for idx in range(4):
            if idx < 2:
                chains.append(dict(
                    col=idx * SC, p_half=pa, p_quar=pb,
                    kb=bit_a, qk=2 * bit_a + bit_b, qs=2 * bit_a + (1 - bit_b),
                ))
            else:
                chains.append(dict(
                    col=idx * SC, p_half=pb, p_quar=pa,
                    kb=bit_b, qk=2 * bit_b + bit1a, qs=2 * bit_b + (1 - bit1a),
                ))

        barrier_sem = pltpu.get_barrier_semaphore()
        for nbr in (pa, pb):
            pl.semaphore_signal(
                barrier_sem, inc=1,
                device_id=(nbr,), device_id_type=pl.DeviceIdType.MESH,
            )
        pl.semaphore_wait(barrier_sem, 2)

        def gemm_block(rh, col):
            out_ref[pl.ds(rh * H, H), pl.ds(col, SC)] = jnp.dot(
                x_ref[pl.ds(rh * H, H), :],
                w_ref[:, pl.ds(col, SC)],
                preferred_element_type=jnp.float32,
            )

        def copy(src, dst, phase, idx, peer):
            return pltpu.make_async_remote_copy(
                src_ref=src, dst_ref=dst,
                send_sem=send_sems.at[phase, idx],
                recv_sem=recv_sems.at[phase, idx],
                device_id=(peer,),
                device_id_type=pl.DeviceIdType.MESH,
            )

        p1 = {}
        for idx in ORDER:
            ch = chains[idx]
            gemm_block(1 - ch["kb"], ch["col"])
            p1[idx] = copy(
                out_ref.at[pl.ds((1 - ch["kb"]) * H, H), pl.ds(ch["col"], SC)],
                c1_ref.at[idx], 0, idx, ch["p_half"])
            p1[idx].start()

        for idx in ORDER:
            gemm_block(chains[idx]["kb"], chains[idx]["col"])

        p2 = {}
        for idx in ORDER:
            ch = chains[idx]
            p1[idx].wait()
            out_ref[pl.ds(ch["kb"] * H, H), pl.ds(ch["col"], SC)] += c1_ref[idx]
            p2[idx] = copy(
                out_ref.at[pl.ds(ch["qs"] * Q, Q), pl.ds(ch["col"], SC)],
                c2_ref.at[idx], 1, idx, ch["p_quar"])
            p2[idx].start()

        p3 = {}
        for idx in ORDER:
            ch = chains[idx]
            p2[idx].wait()
            out_ref[pl.ds(ch["qk"] * Q, Q), pl.ds(ch["col"], SC)] += c2_ref[idx]
            p3[idx] = copy(
                out_ref.at[pl.ds(ch["qk"] * Q, Q), pl.ds(ch["col"], SC)],
                out_ref.at[pl.ds(ch["qk"] * Q, Q), pl.ds(ch["col"], SC)],
                2, idx, ch["p_quar"])
            p3[idx].start()

        p4 = {}
        for idx in ORDER:
            ch = chains[idx]
            p3[idx].wait()
            p4[idx] = copy(
                out_ref.at[pl.ds(ch["kb"] * H, H), pl.ds(ch["col"], SC)],
                out_ref.at[pl.ds(ch["kb"] * H, H), pl.ds(ch["col"], SC)],
                3, idx, ch["p_half"])
            p4[idx].start()

        for idx in ORDER:
            p4[idx].wait()

    return pl.pallas_call(
        body,
        out_shape=jax.ShapeDtypeStruct((M, N), jnp.float32),
        in_specs=[
            pl.BlockSpec(memory_space=pltpu.VMEM),
            pl.BlockSpec(memory_space=pltpu.VMEM),
        ],
        out_specs=pl.BlockSpec(memory_space=pltpu.VMEM),
        scratch_shapes=[
            pltpu.VMEM((4, H, SC), jnp.float32),
            pltpu.VMEM((4, Q, SC), jnp.float32),
            pltpu.SemaphoreType.DMA((4, 4)),
            pltpu.SemaphoreType.DMA((4, 4)),
        ],
        compiler_params=pltpu.CompilerParams(collective_id=0),
    )(x, w_mat)
